# Initial kernel scaffold; baseline (speedup 1.0000x reference)
import jax
import jax.numpy as jnp
from jax.experimental import pallas as pl


def kernel(features, msk, samp_bias1, samp_bias2, W1, W2, Wd, bd, neigh, nodes, perm):
    raise NotImplementedError("write your pallas kernel here")


# speedup vs baseline: 3.7453x; 3.7453x over previous
import functools

import jax
import jax.numpy as jnp
from jax import lax
from jax.experimental import pallas as pl
from jax.experimental.pallas import tpu as pltpu
from jax.experimental.pallas import tpu_sc as plsc

N, D, H, B, S = 50000, 128, 256, 10000, 10

NW = 32
RPW = 1568
NP = NW * RPW
C1 = 32
NCH1 = RPW // C1

BPW = 320
BP = NW * BPW
C2 = 32
NCH2 = BPW // C2

_mesh = plsc.VectorSubcoreMesh(core_axis_name="c", subcore_axis_name="s")


@functools.partial(
    pl.kernel,
    mesh=_mesh,
    out_type=(jax.ShapeDtypeStruct((NP, D), jnp.float32),
              jax.ShapeDtypeStruct((NP, D), jnp.float32)),
    scratch_types=[
        pltpu.VMEM((C1 * S,), jnp.int32),
        pltpu.VMEM((C1 * S,), jnp.int32),
        pltpu.VMEM((C1,), jnp.int32),
        pltpu.VMEM((C1, D), jnp.float32),
        pltpu.VMEM((C1, D), jnp.float32),
        pltpu.VMEM((C1 * S, D), jnp.float32),
        pltpu.VMEM((C1, D), jnp.float32),
        pltpu.VMEM((C1, D), jnp.float32),
        pltpu.SemaphoreType.DMA,
        pltpu.SemaphoreType.DMA,
    ],
)
def _sc_agg1(feat, nflat, perm, outT, outC,
             nb_v, nbc_v, sfc_v, selfT_v, selfC_v, rows_v,
             obT_v, obC_v, sem, sem2):
    wid = lax.axis_index("s") * 2 + lax.axis_index("c")
    base = wid * RPW

    def accum(self_v, ob_v):
        def node(i, _):
            for k in range(D // 16):
                sl = pl.ds(k * 16, 16)
                acc = self_v[i, sl]
                for r in range(S):
                    acc = acc + rows_v[i * S + r, sl]
                ob_v[i, sl] = acc
            return 0
        lax.fori_loop(0, C1, node, 0, unroll=False)

    def chunk(ci, _):
        nb0 = base + ci * C1
        pltpu.sync_copy(nflat.at[pl.ds(nb0 * S, C1 * S)], nb_v)
        cpT = pltpu.async_copy(feat.at[nb_v], rows_v, sem)
        pltpu.sync_copy(feat.at[pl.ds(nb0, C1)], selfT_v)
        pltpu.sync_copy(perm.at[pl.ds(nb0, C1)], sfc_v)
        pltpu.async_copy(perm.at[nb_v], nbc_v, sem2).wait()
        cpT.wait()
        accum(selfT_v, obT_v)
        cpC = pltpu.async_copy(feat.at[nbc_v], rows_v, sem)
        pltpu.async_copy(feat.at[sfc_v], selfC_v, sem2).wait()
        cpC.wait()
        accum(selfC_v, obC_v)
        pltpu.sync_copy(obT_v, outT.at[pl.ds(nb0, C1)])
        pltpu.sync_copy(obC_v, outC.at[pl.ds(nb0, C1)])
        return 0

    lax.fori_loop(0, NCH1, chunk, 0, unroll=False)


@functools.partial(
    pl.kernel,
    mesh=_mesh,
    out_type=(jax.ShapeDtypeStruct((BP, H), jnp.float32),
              jax.ShapeDtypeStruct((BP, H), jnp.float32)),
    scratch_types=[
        pltpu.VMEM((C2 * (S + 1),), jnp.int32),
        pltpu.VMEM((C2 * (S + 1), H), jnp.float32),
        pltpu.VMEM((C2, H), jnp.float32),
        pltpu.VMEM((C2, H), jnp.float32),
        pltpu.SemaphoreType.DMA,
    ],
)
def _sc_agg2(h1, h1c, idxflat, outT, outC, idx_v, rows_v, obT_v, obC_v, sem):
    wid = lax.axis_index("s") * 2 + lax.axis_index("c")
    base = wid * BPW

    def accum(ob_v):
        def node(i, _):
            for k in range(H // 16):
                sl = pl.ds(k * 16, 16)
                acc = rows_v[i * (S + 1), sl]
                for r in range(1, S + 1):
                    acc = acc + rows_v[i * (S + 1) + r, sl]
                ob_v[i, sl] = acc
            return 0
        lax.fori_loop(0, C2, node, 0, unroll=False)

    def chunk(ci, _):
        nb0 = base + ci * C2
        pltpu.sync_copy(idxflat.at[pl.ds(nb0 * (S + 1), C2 * (S + 1))], idx_v)
        pltpu.async_copy(h1.at[idx_v], rows_v, sem).wait()
        accum(obT_v)
        pltpu.async_copy(h1c.at[idx_v], rows_v, sem).wait()
        accum(obC_v)
        pltpu.sync_copy(obT_v, outT.at[pl.ds(nb0, C2)])
        pltpu.sync_copy(obC_v, outC.at[pl.ds(nb0, C2)])
        return 0

    lax.fori_loop(0, NCH2, chunk, 0, unroll=False)


def _mm1_body(aT_ref, aC_ref, w_ref, oT_ref, oC_ref):
    w = w_ref[...]
    dn = (((1,), (1,)), ((), ()))
    oT_ref[...] = jnp.maximum(
        lax.dot_general(aT_ref[...], w, dn, preferred_element_type=jnp.float32), 0.0)
    oC_ref[...] = jnp.maximum(
        lax.dot_general(aC_ref[...], w, dn, preferred_element_type=jnp.float32), 0.0)


_BM = 512


def _tc_mm1(aggT, aggC, W1s):
    nb = NP // _BM
    return pl.pallas_call(
        _mm1_body,
        grid=(nb,),
        in_specs=[
            pl.BlockSpec((_BM, D), lambda i: (i, 0)),
            pl.BlockSpec((_BM, D), lambda i: (i, 0)),
            pl.BlockSpec((H, D), lambda i: (0, 0)),
        ],
        out_specs=[
            pl.BlockSpec((_BM, H), lambda i: (i, 0)),
            pl.BlockSpec((_BM, H), lambda i: (i, 0)),
        ],
        out_shape=[
            jax.ShapeDtypeStruct((NP, H), jnp.float32),
            jax.ShapeDtypeStruct((NP, H), jnp.float32),
        ],
    )(aggT, aggC, W1s)


def _head_body(sT_ref, sC_ref, w2_ref, wd_ref, msk_ref, sb1_ref, sb2_ref,
               o1_ref, o2_ref):
    dn = (((1,), (1,)), ((), ()))
    w2 = w2_ref[...]
    h2 = jnp.maximum(lax.dot_general(sT_ref[...], w2, dn,
                                     preferred_element_type=jnp.float32), 0.0)
    h2c = jnp.maximum(lax.dot_general(sC_ref[...], w2, dn,
                                      preferred_element_type=jnp.float32), 0.0)
    m = msk_ref[...]
    c = jnp.dot(m, h2, preferred_element_type=jnp.float32) / jnp.sum(m)
    c = jax.nn.sigmoid(c)
    cw = lax.dot_general(c, wd_ref[...], dn,
                         preferred_element_type=jnp.float32)
    o1_ref[...] = lax.dot_general(cw, h2, dn,
                                  preferred_element_type=jnp.float32) + sb1_ref[...]
    o2_ref[...] = lax.dot_general(cw, h2c, dn,
                                  preferred_element_type=jnp.float32) + sb2_ref[...]


def _tc_head(sT, sC, W2s, Wd, msk_p, sb1, sb2):
    full = lambda shp: pl.BlockSpec(shp, lambda: (0,) * len(shp))
    return pl.pallas_call(
        _head_body,
        in_specs=[full((BP, H)), full((BP, H)), full((H, H)), full((H, H)),
                  full((1, BP)), full((1, BP)), full((1, BP))],
        out_specs=[full((1, BP)), full((1, BP))],
        out_shape=[jax.ShapeDtypeStruct((1, BP), jnp.float32),
                   jax.ShapeDtypeStruct((1, BP), jnp.float32)],
    )(sT, sC, W2s, Wd, msk_p, sb1, sb2)


@jax.jit
def kernel(features, msk, samp_bias1, samp_bias2, W1, W2, Wd, bd, neigh,
           nodes, perm):
    f32 = jnp.float32
    feat_p = jnp.zeros((NP, D), f32).at[:N].set(features)
    nflat = jnp.zeros((NP, S), jnp.int32).at[:N].set(neigh).reshape(-1)
    perm_p = jnp.zeros((NP,), jnp.int32).at[:N].set(perm)

    aggT, aggC = _sc_agg1(feat_p, nflat, perm_p)
    h1, h1c = _tc_mm1(aggT, aggC, W1 * (1.0 / (S + 1)))

    nodes_p = jnp.zeros((BP,), jnp.int32).at[:B].set(nodes)
    idx2 = jnp.concatenate(
        [nodes_p[:, None], jnp.take(nflat.reshape(NP, S), nodes_p, axis=0)],
        axis=1).reshape(-1)

    s2T, s2C = _sc_agg2(h1, h1c, idx2)

    msk_p = jnp.zeros((1, BP), f32).at[:, :B].set(msk)
    sb1 = jnp.zeros((1, BP), f32).at[:, :B].set(samp_bias1 + bd)
    sb2 = jnp.zeros((1, BP), f32).at[:, :B].set(samp_bias2 + bd)

    o1, o2 = _tc_head(s2T, s2C, W2 * (1.0 / (S + 1)), Wd, msk_p, sb1, sb2)
    return jnp.concatenate([o1[:, :B], o2[:, :B]], axis=1)


# speedup vs baseline: 6.6222x; 1.7681x over previous
import functools

import jax
import jax.numpy as jnp
from jax import lax
from jax.experimental import pallas as pl
from jax.experimental.pallas import tpu as pltpu
from jax.experimental.pallas import tpu_sc as plsc

N, D, H, B, S = 50000, 128, 256, 10000, 10
SS = S + 1

NW = 32
RPW = 1568
NP = NW * RPW
C1 = 16
NCH1 = RPW // C1

BPW = 320
BP = NW * BPW
C2 = 16
NCH2 = BPW // C2

_mesh = plsc.VectorSubcoreMesh(core_axis_name="c", subcore_axis_name="s")


def _accum(rows_v, ob_v, c, width):
    @plsc.parallel_loop(0, c, unroll=2)
    def _(i):
        for k in range(width // 16):
            sl = pl.ds(k * 16, 16)
            acc = rows_v[i * SS, sl]
            for r in range(1, SS):
                acc = acc + rows_v[i * SS + r, sl]
            ob_v[i, sl] = acc


@functools.partial(
    pl.kernel,
    mesh=_mesh,
    out_type=(jax.ShapeDtypeStruct((NP, D), jnp.float32),
              jax.ShapeDtypeStruct((NP, D), jnp.float32)),
    scratch_types=[
        pltpu.VMEM((RPW * SS,), jnp.int32),
        pltpu.VMEM((RPW * SS,), jnp.int32),
        pltpu.VMEM((C1 * SS, D), jnp.float32),
        pltpu.VMEM((C1 * SS, D), jnp.float32),
        pltpu.VMEM((C1, D), jnp.float32),
        pltpu.VMEM((C1, D), jnp.float32),
        pltpu.SemaphoreType.DMA,
        pltpu.SemaphoreType.DMA,
        pltpu.SemaphoreType.DMA,
        pltpu.SemaphoreType.DMA,
        pltpu.SemaphoreType.DMA,
    ],
)
def _sc_agg1(feat, idxflat, perm, outT, outC,
             nb_all, nbc_all, rowsT_v, rowsC_v, obT_v, obC_v,
             sem_rT, sem_rC, sem_oT, sem_oC, sem_g):
    wid = lax.axis_index("s") * 2 + lax.axis_index("c")
    base = wid * RPW

    pltpu.sync_copy(idxflat.at[pl.ds(base * SS, RPW * SS)], nb_all)
    cp_nbc = pltpu.async_copy(perm.at[nb_all], nbc_all, sem_g)

    def start(idx_all, rows_v, sem, ci):
        pltpu.async_copy(feat.at[idx_all.at[pl.ds(ci * C1 * SS, C1 * SS)]],
                         rows_v, sem)

    start(nb_all, rowsT_v, sem_rT, 0)
    cp_nbc.wait()
    start(nbc_all, rowsC_v, sem_rC, 0)

    def phase(ci, idx_all, rows_v, sem_r, ob_v, sem_o, out_hbm):
        nb0 = base + ci * C1

        @pl.when(ci > 0)
        def _():
            pltpu.make_async_copy(ob_v, out_hbm.at[pl.ds(nb0 - C1, C1)],
                                  sem_o).wait()
        pltpu.make_async_copy(
            feat.at[idx_all.at[pl.ds(ci * C1 * SS, C1 * SS)]],
            rows_v, sem_r).wait()
        _accum(rows_v, ob_v, C1, D)
        pltpu.async_copy(ob_v, out_hbm.at[pl.ds(nb0, C1)], sem_o)

        @pl.when(ci + 1 < NCH1)
        def _():
            start(idx_all, rows_v, sem_r, ci + 1)

    def chunk(ci, _):
        phase(ci, nb_all, rowsT_v, sem_rT, obT_v, sem_oT, outT)
        phase(ci, nbc_all, rowsC_v, sem_rC, obC_v, sem_oC, outC)
        return 0

    lax.fori_loop(0, NCH1, chunk, 0, unroll=False)
    last = base + (NCH1 - 1) * C1
    pltpu.make_async_copy(obT_v, outT.at[pl.ds(last, C1)], sem_oT).wait()
    pltpu.make_async_copy(obC_v, outC.at[pl.ds(last, C1)], sem_oC).wait()


@functools.partial(
    pl.kernel,
    mesh=_mesh,
    out_type=(jax.ShapeDtypeStruct((BP, H), jnp.float32),
              jax.ShapeDtypeStruct((BP, H), jnp.float32)),
    scratch_types=[
        pltpu.VMEM((BPW * SS,), jnp.int32),
        pltpu.VMEM((C2 * SS, H), jnp.float32),
        pltpu.VMEM((C2 * SS, H), jnp.float32),
        pltpu.VMEM((C2, H), jnp.float32),
        pltpu.VMEM((C2, H), jnp.float32),
        pltpu.SemaphoreType.DMA,
        pltpu.SemaphoreType.DMA,
        pltpu.SemaphoreType.DMA,
        pltpu.SemaphoreType.DMA,
    ],
)
def _sc_agg2(h1, h1c, idxflat, outT, outC,
             idx_all, rowsT_v, rowsC_v, obT_v, obC_v,
             sem_rT, sem_rC, sem_oT, sem_oC):
    wid = lax.axis_index("s") * 2 + lax.axis_index("c")
    base = wid * BPW

    pltpu.sync_copy(idxflat.at[pl.ds(base * SS, BPW * SS)], idx_all)

    def start(tab, rows_v, sem, ci):
        pltpu.async_copy(tab.at[idx_all.at[pl.ds(ci * C2 * SS, C2 * SS)]],
                         rows_v, sem)

    start(h1, rowsT_v, sem_rT, 0)
    start(h1c, rowsC_v, sem_rC, 0)

    def phase(ci, tab, rows_v, sem_r, ob_v, sem_o, out_hbm):
        nb0 = base + ci * C2

        @pl.when(ci > 0)
        def _():
            pltpu.make_async_copy(ob_v, out_hbm.at[pl.ds(nb0 - C2, C2)],
                                  sem_o).wait()
        pltpu.make_async_copy(
            tab.at[idx_all.at[pl.ds(ci * C2 * SS, C2 * SS)]],
            rows_v, sem_r).wait()
        _accum(rows_v, ob_v, C2, H)
        pltpu.async_copy(ob_v, out_hbm.at[pl.ds(nb0, C2)], sem_o)

        @pl.when(ci + 1 < NCH2)
        def _():
            start(tab, rows_v, sem_r, ci + 1)

    def chunk(ci, _):
        phase(ci, h1, rowsT_v, sem_rT, obT_v, sem_oT, outT)
        phase(ci, h1c, rowsC_v, sem_rC, obC_v, sem_oC, outC)
        return 0

    lax.fori_loop(0, NCH2, chunk, 0, unroll=False)
    last = base + (NCH2 - 1) * C2
    pltpu.make_async_copy(obT_v, outT.at[pl.ds(last, C2)], sem_oT).wait()
    pltpu.make_async_copy(obC_v, outC.at[pl.ds(last, C2)], sem_oC).wait()


def _mm1_body(aT_ref, aC_ref, w_ref, oT_ref, oC_ref):
    w = w_ref[...]
    dn = (((1,), (1,)), ((), ()))
    oT_ref[...] = jnp.maximum(
        lax.dot_general(aT_ref[...], w, dn, preferred_element_type=jnp.float32), 0.0)
    oC_ref[...] = jnp.maximum(
        lax.dot_general(aC_ref[...], w, dn, preferred_element_type=jnp.float32), 0.0)


_BM = 512


def _tc_mm1(aggT, aggC, W1s):
    nb = NP // _BM
    return pl.pallas_call(
        _mm1_body,
        grid=(nb,),
        in_specs=[
            pl.BlockSpec((_BM, D), lambda i: (i, 0)),
            pl.BlockSpec((_BM, D), lambda i: (i, 0)),
            pl.BlockSpec((H, D), lambda i: (0, 0)),
        ],
        out_specs=[
            pl.BlockSpec((_BM, H), lambda i: (i, 0)),
            pl.BlockSpec((_BM, H), lambda i: (i, 0)),
        ],
        out_shape=[
            jax.ShapeDtypeStruct((NP, H), jnp.float32),
            jax.ShapeDtypeStruct((NP, H), jnp.float32),
        ],
    )(aggT, aggC, W1s)


def _head_body(sT_ref, sC_ref, w2_ref, wd_ref, msk_ref, sb1_ref, sb2_ref,
               o1_ref, o2_ref):
    dn = (((1,), (1,)), ((), ()))
    w2 = w2_ref[...]
    h2 = jnp.maximum(lax.dot_general(sT_ref[...], w2, dn,
                                     preferred_element_type=jnp.float32), 0.0)
    h2c = jnp.maximum(lax.dot_general(sC_ref[...], w2, dn,
                                      preferred_element_type=jnp.float32), 0.0)
    m = msk_ref[...]
    c = jnp.dot(m, h2, preferred_element_type=jnp.float32) / jnp.sum(m)
    c = jax.nn.sigmoid(c)
    cw = lax.dot_general(c, wd_ref[...], dn,
                         preferred_element_type=jnp.float32)
    o1_ref[...] = lax.dot_general(cw, h2, dn,
                                  preferred_element_type=jnp.float32) + sb1_ref[...]
    o2_ref[...] = lax.dot_general(cw, h2c, dn,
                                  preferred_element_type=jnp.float32) + sb2_ref[...]


def _tc_head(sT, sC, W2s, Wd, msk_p, sb1, sb2):
    full = lambda shp: pl.BlockSpec(shp, lambda: (0,) * len(shp))
    return pl.pallas_call(
        _head_body,
        in_specs=[full((BP, H)), full((BP, H)), full((H, H)), full((H, H)),
                  full((1, BP)), full((1, BP)), full((1, BP))],
        out_specs=[full((1, BP)), full((1, BP))],
        out_shape=[jax.ShapeDtypeStruct((1, BP), jnp.float32),
                   jax.ShapeDtypeStruct((1, BP), jnp.float32)],
    )(sT, sC, W2s, Wd, msk_p, sb1, sb2)


@jax.jit
def kernel(features, msk, samp_bias1, samp_bias2, W1, W2, Wd, bd, neigh,
           nodes, perm):
    f32 = jnp.float32
    idxT = jnp.concatenate(
        [jnp.arange(N, dtype=jnp.int32)[:, None], neigh], axis=1)
    idxT_p = jnp.zeros((NP, SS), jnp.int32).at[:N].set(idxT).reshape(-1)

    aggT, aggC = _sc_agg1(features, idxT_p, perm)
    h1, h1c = _tc_mm1(aggT, aggC, W1 * (1.0 / SS))

    nodes_p = jnp.zeros((BP,), jnp.int32).at[:B].set(nodes)
    idx2 = jnp.concatenate(
        [nodes_p[:, None], jnp.take(neigh, nodes_p, axis=0)], axis=1).reshape(-1)

    s2T, s2C = _sc_agg2(h1, h1c, idx2)

    msk_p = jnp.zeros((1, BP), f32).at[:, :B].set(msk)
    sb1 = jnp.zeros((1, BP), f32).at[:, :B].set(samp_bias1 + bd)
    sb2 = jnp.zeros((1, BP), f32).at[:, :B].set(samp_bias2 + bd)

    o1, o2 = _tc_head(s2T, s2C, W2 * (1.0 / SS), Wd, msk_p, sb1, sb2)
    return jnp.concatenate([o1[:, :B], o2[:, :B]], axis=1)
